# Initial kernel scaffold; baseline (speedup 1.0000x reference)
import jax
import jax.numpy as jnp
from jax.experimental import pallas as pl


def kernel(input_embs, seg_ids, seg_table, ln_gamma, ln_beta):
    raise NotImplementedError("write your pallas kernel here")


# speedup vs baseline: 4.0447x; 4.0447x over previous
import jax
import jax.numpy as jnp
from jax.experimental import pallas as pl

_HIDDEN = 768
_EPS = 1e-12
_T = 1024


def _fused_kernel(ids_ref, x_ref, table_ref, gamma_ref, beta_ref, out_ref):
    ids = ids_ref[0, 0, :]
    x = x_ref[...]
    table = table_ref[...]
    onehot = (ids[:, None] == jax.lax.broadcasted_iota(jnp.int32, (_T, 4), 1))
    seg = jnp.dot(onehot.astype(jnp.float32), table,
                  preferred_element_type=jnp.float32)
    e = x + seg
    mean = jnp.mean(e, axis=1, keepdims=True)
    d = e - mean
    var = jnp.mean(d * d, axis=1, keepdims=True)
    normed = d * jax.lax.rsqrt(var + _EPS)
    out_ref[...] = normed * gamma_ref[...] + beta_ref[...]


def kernel(input_embs, seg_ids, seg_table, ln_gamma, ln_beta):
    b, s, h = input_embs.shape
    n_tok = b * s
    nb = n_tok // _T
    x = input_embs.reshape(n_tok, h)
    ids = seg_ids.astype(jnp.int32).reshape(nb, 1, _T)
    gamma = ln_gamma.reshape(1, h)
    beta = ln_beta.reshape(1, h)

    out = pl.pallas_call(
        _fused_kernel,
        grid=(nb,),
        in_specs=[
            pl.BlockSpec((1, 1, _T), lambda i: (i, 0, 0)),
            pl.BlockSpec((_T, h), lambda i: (i, 0)),
            pl.BlockSpec((4, h), lambda i: (0, 0)),
            pl.BlockSpec((1, h), lambda i: (0, 0)),
            pl.BlockSpec((1, h), lambda i: (0, 0)),
        ],
        out_specs=pl.BlockSpec((_T, h), lambda i: (i, 0)),
        out_shape=jax.ShapeDtypeStruct((n_tok, h), jnp.float32),
    )(ids, x, seg_table, gamma, beta)
    return out.reshape(b, s, h)


# speedup vs baseline: 4.5151x; 1.1163x over previous
import jax
import jax.numpy as jnp
from jax.experimental import pallas as pl

_HIDDEN = 768
_EPS = 1e-12
_T = 2048


def _fused_kernel(ids_ref, x_ref, table_ref, gamma_ref, beta_ref, out_ref):
    ids = ids_ref[0, 0, :]
    x = x_ref[...]
    table = table_ref[...]
    onehot = (ids[:, None] == jax.lax.broadcasted_iota(jnp.int32, (_T, 4), 1))
    seg = jnp.dot(onehot.astype(jnp.float32), table,
                  preferred_element_type=jnp.float32)
    e = x + seg
    mean = jnp.mean(e, axis=1, keepdims=True)
    d = e - mean
    var = jnp.mean(d * d, axis=1, keepdims=True)
    normed = d * jax.lax.rsqrt(var + _EPS)
    out_ref[...] = normed * gamma_ref[...] + beta_ref[...]


def kernel(input_embs, seg_ids, seg_table, ln_gamma, ln_beta):
    b, s, h = input_embs.shape
    n_tok = b * s
    nb = n_tok // _T
    x = input_embs.reshape(n_tok, h)
    ids = seg_ids.astype(jnp.int32).reshape(nb, 1, _T)
    gamma = ln_gamma.reshape(1, h)
    beta = ln_beta.reshape(1, h)

    out = pl.pallas_call(
        _fused_kernel,
        grid=(nb,),
        in_specs=[
            pl.BlockSpec((1, 1, _T), lambda i: (i, 0, 0)),
            pl.BlockSpec((_T, h), lambda i: (i, 0)),
            pl.BlockSpec((4, h), lambda i: (0, 0)),
            pl.BlockSpec((1, h), lambda i: (0, 0)),
            pl.BlockSpec((1, h), lambda i: (0, 0)),
        ],
        out_specs=pl.BlockSpec((_T, h), lambda i: (i, 0)),
        out_shape=jax.ShapeDtypeStruct((n_tok, h), jnp.float32),
    )(ids, x, seg_table, gamma, beta)
    return out.reshape(b, s, h)


# speedup vs baseline: 4.5551x; 1.0089x over previous
import jax
import jax.numpy as jnp
from jax.experimental import pallas as pl

_HIDDEN = 768
_EPS = 1e-12
_T = 4096


def _fused_kernel(ids_ref, x_ref, table_ref, gamma_ref, beta_ref, out_ref):
    ids = ids_ref[0, 0, :]
    x = x_ref[...]
    table = table_ref[...]
    onehot = (ids[:, None] == jax.lax.broadcasted_iota(jnp.int32, (_T, 4), 1))
    seg = jnp.dot(onehot.astype(jnp.float32), table,
                  preferred_element_type=jnp.float32)
    e = x + seg
    mean = jnp.mean(e, axis=1, keepdims=True)
    d = e - mean
    var = jnp.mean(d * d, axis=1, keepdims=True)
    normed = d * jax.lax.rsqrt(var + _EPS)
    out_ref[...] = normed * gamma_ref[...] + beta_ref[...]


def kernel(input_embs, seg_ids, seg_table, ln_gamma, ln_beta):
    b, s, h = input_embs.shape
    n_tok = b * s
    nb = n_tok // _T
    x = input_embs.reshape(n_tok, h)
    ids = seg_ids.astype(jnp.int32).reshape(nb, 1, _T)
    gamma = ln_gamma.reshape(1, h)
    beta = ln_beta.reshape(1, h)

    out = pl.pallas_call(
        _fused_kernel,
        grid=(nb,),
        in_specs=[
            pl.BlockSpec((1, 1, _T), lambda i: (i, 0, 0)),
            pl.BlockSpec((_T, h), lambda i: (i, 0)),
            pl.BlockSpec((4, h), lambda i: (0, 0)),
            pl.BlockSpec((1, h), lambda i: (0, 0)),
            pl.BlockSpec((1, h), lambda i: (0, 0)),
        ],
        out_specs=pl.BlockSpec((_T, h), lambda i: (i, 0)),
        out_shape=jax.ShapeDtypeStruct((n_tok, h), jnp.float32),
    )(ids, x, seg_table, gamma, beta)
    return out.reshape(b, s, h)
